# baseline (speedup 1.0000x reference)
import functools

import jax
import jax.numpy as jnp
from jax import lax
from jax.experimental import pallas as pl
from jax.experimental.pallas import tpu as pltpu
from jax.experimental.pallas import tpu_sc as plsc

N = 10000
D = 128
E = 320000
NC = 2
NS = 16
NW = NC * NS
EPW = E // NW
C = 40
NCHUNK = EPW // C
SS = 25
NSUPER = NCHUNK // SS
SEDGE = SS * C
N_PAD = 10240
STRIP = N_PAD // NS


NBUF = 8


def _sc_scatter(x0, x1, ei00, ei01, zstrip):
    mesh = plsc.VectorSubcoreMesh(core_axis_name="c", subcore_axis_name="s")

    @functools.partial(
        pl.kernel,
        out_type=jax.ShapeDtypeStruct((3, NC, N_PAD, D), jnp.float32),
        mesh=mesh,
        scratch_types=(
            [pltpu.VMEM((SEDGE,), jnp.int32)] * 2
            + [pltpu.VMEM((NBUF * C, D), jnp.float32)]
            + [pltpu.SemaphoreType.DMA((NBUF,))] * 2
            + [pltpu.VMEM_SHARED((N_PAD, D), jnp.float32)]
        ),
    )
    def k(x0_hbm, x1_hbm, ei00_hbm, ei01_hbm, z_hbm, p_hbm, *scr):
        gidx, sidx = scr[0], scr[1]
        rows = scr[2]
        gsem, ssem = scr[3], scr[4]
        acc = scr[5]
        cid = lax.axis_index("c")
        sid = lax.axis_index("s")
        wid = sid * NC + cid
        ebase = wid * EPW
        rlo = sid * STRIP

        def zero_strip():
            pltpu.sync_copy(z_hbm, acc.at[pl.ds(rlo, STRIP)])

        def run_pass(ei_hbm, g_sel, s_sel, table_hbm):
            def sbody(s, scarry):
                sbase = ebase + s * SEDGE
                pltpu.sync_copy(ei_hbm.at[pl.ds(g_sel * E + sbase, SEDGE)],
                                gidx)
                pltpu.sync_copy(ei_hbm.at[pl.ds(s_sel * E + sbase, SEDGE)],
                                sidx)

                def gather_desc(c, slot):
                    return pltpu.make_async_copy(
                        table_hbm.at[gidx.at[pl.ds(c * C, C)]],
                        rows.at[pl.ds(slot * C, C)], gsem.at[slot])

                def scatter_start(c, slot):
                    pltpu.async_copy(rows.at[pl.ds(slot * C, C)],
                                     acc.at[sidx.at[pl.ds(c * C, C)]],
                                     ssem.at[slot], add=True)

                def scatter_wait(c, slot):
                    pltpu.make_async_copy(
                        rows.at[pl.ds(slot * C, C)],
                        acc.at[sidx.at[pl.ds(c * C, C)]],
                        ssem.at[slot]).wait()

                K = NBUF - 1
                for b in range(K):
                    gather_desc(b, b).start()

                def body(c, carry):
                    slot = lax.rem(c, NBUF)
                    nslot = lax.rem(c + K, NBUF)

                    @pl.when(c + K < SS)
                    def _():
                        @pl.when(c >= 1)
                        def _():
                            scatter_wait(c - 1, nslot)
                        gather_desc(c + K, nslot).start()

                    gather_desc(c, slot).wait()
                    scatter_start(c, slot)
                    return carry
                lax.fori_loop(0, SS, body, 0)
                for b in range(NBUF):
                    cc = SS - NBUF + b
                    scatter_wait(cc, cc % NBUF)
                return scarry
            lax.fori_loop(0, NSUPER, sbody, 0)

        def dump(phase):
            pltpu.sync_copy(acc.at[pl.ds(rlo, STRIP)],
                            p_hbm.at[phase, cid, pl.ds(rlo, STRIP)])

        zero_strip()
        plsc.subcore_barrier()
        run_pass(ei00_hbm, 1, 0, x0_hbm)
        run_pass(ei00_hbm, 0, 1, x0_hbm)
        plsc.subcore_barrier()
        dump(0)
        zero_strip()
        plsc.subcore_barrier()
        run_pass(ei01_hbm, 1, 0, x1_hbm)
        plsc.subcore_barrier()
        dump(1)
        zero_strip()
        plsc.subcore_barrier()
        run_pass(ei01_hbm, 0, 1, x0_hbm)
        plsc.subcore_barrier()
        dump(2)

    return k(x0, x1, ei00, ei01, zstrip)


def _combine(x0, x1, P, W0, W1, b0, b1):
    BR = 1000
    grid = (N // BR,)

    def body(x0_ref, x1_ref, p_ref, w0_ref, w1_ref, b0_ref, b1_ref,
             o0_ref, o1_ref):
        p = p_ref[...]
        a00 = p[0, 0] + p[0, 1]
        a01 = p[1, 0] + p[1, 1]
        a10 = p[2, 0] + p[2, 1]
        w0 = w0_ref[...]
        w1 = w1_ref[...]
        u0 = x0_ref[...] + a00
        o0_ref[...] = (
            jnp.dot(u0, w0, preferred_element_type=jnp.float32)
            + jnp.dot(a01, w1, preferred_element_type=jnp.float32)
            + 3.0 * b0_ref[...] + b1_ref[...]
        )
        o1_ref[...] = (
            jnp.dot(x1_ref[...] , w1, preferred_element_type=jnp.float32)
            + jnp.dot(a10, w0, preferred_element_type=jnp.float32)
            + b0_ref[...] + b1_ref[...]
        )

    return pl.pallas_call(
        body,
        grid=grid,
        in_specs=[
            pl.BlockSpec((BR, D), lambda i: (i, 0)),
            pl.BlockSpec((BR, D), lambda i: (i, 0)),
            pl.BlockSpec((3, NC, BR, D), lambda i: (0, 0, i, 0)),
            pl.BlockSpec((D, D), lambda i: (0, 0)),
            pl.BlockSpec((D, D), lambda i: (0, 0)),
            pl.BlockSpec((1, D), lambda i: (0, 0)),
            pl.BlockSpec((1, D), lambda i: (0, 0)),
        ],
        out_specs=[
            pl.BlockSpec((BR, D), lambda i: (i, 0)),
            pl.BlockSpec((BR, D), lambda i: (i, 0)),
        ],
        out_shape=[
            jax.ShapeDtypeStruct((N, D), jnp.float32),
            jax.ShapeDtypeStruct((N, D), jnp.float32),
        ],
    )(x0, x1, P, W0, W1, b0.reshape(1, D), b1.reshape(1, D))


def kernel(x0, x1, edge_index_00, edge_index_01, W0, b0, W1, b1):
    zstrip = jnp.zeros((STRIP, D), jnp.float32)
    P = _sc_scatter(x0, x1, edge_index_00.reshape(-1),
                    edge_index_01.reshape(-1), zstrip)
    out0, out1 = _combine(x0, x1, P, W0, W1, b0, b1)
    return out0, out1


# speedup vs baseline: 1.0998x; 1.0998x over previous
import functools

import jax
import jax.numpy as jnp
from jax import lax
from jax.experimental import pallas as pl
from jax.experimental.pallas import tpu as pltpu
from jax.experimental.pallas import tpu_sc as plsc

N = 10000
D = 128
E = 320000
NC = 2
NS = 16
NW = NC * NS
EPW = E // NW
C = 80
NCHUNK = EPW // C
SS = 25
NSUPER = NCHUNK // SS
SEDGE = SS * C
N_PAD = 10240
STRIP = N_PAD // NS


NBUF = 4


def _sc_scatter(x0, x1, ei00, ei01, zstrip):
    mesh = plsc.VectorSubcoreMesh(core_axis_name="c", subcore_axis_name="s")

    @functools.partial(
        pl.kernel,
        out_type=jax.ShapeDtypeStruct((3, NC, N_PAD, D), jnp.float32),
        mesh=mesh,
        scratch_types=(
            [pltpu.VMEM((SEDGE,), jnp.int32)] * 2
            + [pltpu.VMEM((NBUF * C, D), jnp.float32)]
            + [pltpu.SemaphoreType.DMA((NBUF,))] * 2
            + [pltpu.VMEM_SHARED((N_PAD, D), jnp.float32)]
        ),
    )
    def k(x0_hbm, x1_hbm, ei00_hbm, ei01_hbm, z_hbm, p_hbm, *scr):
        gidx, sidx = scr[0], scr[1]
        rows = scr[2]
        gsem, ssem = scr[3], scr[4]
        acc = scr[5]
        cid = lax.axis_index("c")
        sid = lax.axis_index("s")
        wid = sid * NC + cid
        ebase = wid * EPW
        rlo = sid * STRIP

        def zero_strip():
            pltpu.sync_copy(z_hbm, acc.at[pl.ds(rlo, STRIP)])

        def run_pass(ei_hbm, g_sel, s_sel, table_hbm):
            def sbody(s, scarry):
                sbase = ebase + s * SEDGE
                pltpu.sync_copy(ei_hbm.at[pl.ds(g_sel * E + sbase, SEDGE)],
                                gidx)
                pltpu.sync_copy(ei_hbm.at[pl.ds(s_sel * E + sbase, SEDGE)],
                                sidx)

                def gather_desc(c, slot):
                    return pltpu.make_async_copy(
                        table_hbm.at[gidx.at[pl.ds(c * C, C)]],
                        rows.at[pl.ds(slot * C, C)], gsem.at[slot])

                def scatter_start(c, slot):
                    pltpu.async_copy(rows.at[pl.ds(slot * C, C)],
                                     acc.at[sidx.at[pl.ds(c * C, C)]],
                                     ssem.at[slot], add=True)

                def scatter_wait(c, slot):
                    pltpu.make_async_copy(
                        rows.at[pl.ds(slot * C, C)],
                        acc.at[sidx.at[pl.ds(c * C, C)]],
                        ssem.at[slot]).wait()

                K = NBUF - 1
                for b in range(K):
                    gather_desc(b, b).start()

                def body(c, carry):
                    slot = lax.rem(c, NBUF)
                    nslot = lax.rem(c + K, NBUF)

                    @pl.when(c + K < SS)
                    def _():
                        @pl.when(c >= 1)
                        def _():
                            scatter_wait(c - 1, nslot)
                        gather_desc(c + K, nslot).start()

                    gather_desc(c, slot).wait()
                    scatter_start(c, slot)
                    return carry
                lax.fori_loop(0, SS, body, 0)
                for b in range(NBUF):
                    cc = SS - NBUF + b
                    scatter_wait(cc, cc % NBUF)
                return scarry
            lax.fori_loop(0, NSUPER, sbody, 0)

        def dump(phase):
            pltpu.sync_copy(acc.at[pl.ds(rlo, STRIP)],
                            p_hbm.at[phase, cid, pl.ds(rlo, STRIP)])

        zero_strip()
        plsc.subcore_barrier()
        run_pass(ei00_hbm, 1, 0, x0_hbm)
        run_pass(ei00_hbm, 0, 1, x0_hbm)
        plsc.subcore_barrier()
        dump(0)
        zero_strip()
        plsc.subcore_barrier()
        run_pass(ei01_hbm, 1, 0, x1_hbm)
        plsc.subcore_barrier()
        dump(1)
        zero_strip()
        plsc.subcore_barrier()
        run_pass(ei01_hbm, 0, 1, x0_hbm)
        plsc.subcore_barrier()
        dump(2)

    return k(x0, x1, ei00, ei01, zstrip)


def _combine(x0, x1, P, W0, W1, b0, b1):
    BR = 1000
    grid = (N // BR,)

    def body(x0_ref, x1_ref, p_ref, w0_ref, w1_ref, b0_ref, b1_ref,
             o0_ref, o1_ref):
        p = p_ref[...]
        a00 = p[0, 0] + p[0, 1]
        a01 = p[1, 0] + p[1, 1]
        a10 = p[2, 0] + p[2, 1]
        w0 = w0_ref[...]
        w1 = w1_ref[...]
        u0 = x0_ref[...] + a00
        o0_ref[...] = (
            jnp.dot(u0, w0, preferred_element_type=jnp.float32)
            + jnp.dot(a01, w1, preferred_element_type=jnp.float32)
            + 3.0 * b0_ref[...] + b1_ref[...]
        )
        o1_ref[...] = (
            jnp.dot(x1_ref[...] , w1, preferred_element_type=jnp.float32)
            + jnp.dot(a10, w0, preferred_element_type=jnp.float32)
            + b0_ref[...] + b1_ref[...]
        )

    return pl.pallas_call(
        body,
        grid=grid,
        in_specs=[
            pl.BlockSpec((BR, D), lambda i: (i, 0)),
            pl.BlockSpec((BR, D), lambda i: (i, 0)),
            pl.BlockSpec((3, NC, BR, D), lambda i: (0, 0, i, 0)),
            pl.BlockSpec((D, D), lambda i: (0, 0)),
            pl.BlockSpec((D, D), lambda i: (0, 0)),
            pl.BlockSpec((1, D), lambda i: (0, 0)),
            pl.BlockSpec((1, D), lambda i: (0, 0)),
        ],
        out_specs=[
            pl.BlockSpec((BR, D), lambda i: (i, 0)),
            pl.BlockSpec((BR, D), lambda i: (i, 0)),
        ],
        out_shape=[
            jax.ShapeDtypeStruct((N, D), jnp.float32),
            jax.ShapeDtypeStruct((N, D), jnp.float32),
        ],
    )(x0, x1, P, W0, W1, b0.reshape(1, D), b1.reshape(1, D))


def kernel(x0, x1, edge_index_00, edge_index_01, W0, b0, W1, b1):
    zstrip = jnp.zeros((STRIP, D), jnp.float32)
    P = _sc_scatter(x0, x1, edge_index_00.reshape(-1),
                    edge_index_01.reshape(-1), zstrip)
    out0, out1 = _combine(x0, x1, P, W0, W1, b0, b1)
    return out0, out1


# speedup vs baseline: 1.2808x; 1.1646x over previous
import functools

import jax
import jax.numpy as jnp
from jax import lax
from jax.experimental import pallas as pl
from jax.experimental.pallas import tpu as pltpu
from jax.experimental.pallas import tpu_sc as plsc

N = 10000
D = 128
E = 320000
NC = 2
NS = 16
NW = NC * NS
EPW = E // NW
C = 80
NCHUNK = EPW // C
SS = 25
NSUPER = NCHUNK // SS
SEDGE = SS * C
NBUF = 4
N_PAD = 10112
STRIP = N_PAD // NS


def _sc_scatter(x0, x1, ei00, ei01, zstrip):
    mesh = plsc.VectorSubcoreMesh(core_axis_name="c", subcore_axis_name="s")

    @functools.partial(
        pl.kernel,
        out_type=jax.ShapeDtypeStruct((3, NC, N_PAD, D), jnp.float32),
        mesh=mesh,
        scratch_types=(
            [pltpu.VMEM((2 * SEDGE,), jnp.int32)] * 2
            + [pltpu.VMEM((NBUF * C, D), jnp.float32)]
            + [pltpu.SemaphoreType.DMA((NBUF,))] * 2
            + [pltpu.SemaphoreType.DMA] * 2
            + [pltpu.VMEM_SHARED((N_PAD, D), jnp.float32)]
        ),
    )
    def k(x0_hbm, x1_hbm, ei00_hbm, ei01_hbm, z_hbm, p_hbm, *scr):
        gidx, sidx = scr[0], scr[1]
        rows = scr[2]
        gsem, ssem = scr[3], scr[4]
        gisem, sisem = scr[5], scr[6]
        acc = scr[7]
        cid = lax.axis_index("c")
        sid = lax.axis_index("s")
        wid = sid * NC + cid
        ebase = wid * EPW
        rlo = sid * STRIP

        def zero_strip():
            pltpu.sync_copy(z_hbm, acc.at[pl.ds(rlo, STRIP)])

        def run_pass(ei_hbm, g_sel, s_sel, table_hbm):
            goff = g_sel * E + ebase
            soff = s_sel * E + ebase

            def idx_descs(s2):
                h = lax.rem(s2, 2) * SEDGE
                gd = pltpu.make_async_copy(
                    ei_hbm.at[pl.ds(goff + s2 * SEDGE, SEDGE)],
                    gidx.at[pl.ds(h, SEDGE)], gisem)
                sd = pltpu.make_async_copy(
                    ei_hbm.at[pl.ds(soff + s2 * SEDGE, SEDGE)],
                    sidx.at[pl.ds(h, SEDGE)], sisem)
                return gd, sd

            def ioff(j):
                s2 = lax.div(j, SS)
                return lax.rem(s2, 2) * SEDGE + (j - s2 * SS) * C

            def gather_desc(c, slot):
                return pltpu.make_async_copy(
                    table_hbm.at[gidx.at[pl.ds(ioff(c), C)]],
                    rows.at[pl.ds(slot * C, C)], gsem.at[slot])

            def scatter_start(c, slot):
                pltpu.async_copy(rows.at[pl.ds(slot * C, C)],
                                 acc.at[sidx.at[pl.ds(ioff(c), C)]],
                                 ssem.at[slot], add=True)

            def scatter_wait(c, slot):
                pltpu.make_async_copy(
                    rows.at[pl.ds(slot * C, C)],
                    acc.at[sidx.at[pl.ds(ioff(c), C)]],
                    ssem.at[slot]).wait()

            K = NBUF - 1

            gd, sd = idx_descs(0)
            gd.start()
            sd.start()
            gd.wait()
            sd.wait()
            gd, sd = idx_descs(1)
            gd.start()
            sd.start()
            for b in range(K):
                gather_desc(b, b).start()

            def body(c, carry):
                slot = lax.rem(c, NBUF)
                nslot = lax.rem(c + K, NBUF)

                @pl.when(c + K < NCHUNK)
                def _():
                    @pl.when(c >= 1)
                    def _():
                        scatter_wait(c - 1, nslot)

                    @pl.when((lax.rem(c, SS) == 0) & (c > 0))
                    def _():
                        s2 = lax.div(c, SS) + 1

                        @pl.when(s2 < NSUPER)
                        def _():
                            gd2, sd2 = idx_descs(s2)
                            gd2.start()
                            sd2.start()

                    @pl.when(lax.rem(c + K, SS) == 0)
                    def _():
                        gd2, sd2 = idx_descs(lax.div(c + K, SS))
                        gd2.wait()
                        sd2.wait()
                    gather_desc(c + K, nslot).start()

                gather_desc(c, slot).wait()
                scatter_start(c, slot)
                return carry
            lax.fori_loop(0, NCHUNK, body, 0)
            for b in range(NBUF):
                cc = NCHUNK - NBUF + b
                scatter_wait(cc, cc % NBUF)

        def dump(phase):
            pltpu.sync_copy(acc.at[pl.ds(rlo, STRIP)],
                            p_hbm.at[phase, cid, pl.ds(rlo, STRIP)])

        zero_strip()
        plsc.subcore_barrier()
        run_pass(ei00_hbm, 1, 0, x0_hbm)
        run_pass(ei00_hbm, 0, 1, x0_hbm)
        plsc.subcore_barrier()
        dump(0)
        plsc.subcore_barrier()
        run_pass(ei01_hbm, 1, 0, x1_hbm)
        plsc.subcore_barrier()
        dump(1)
        plsc.subcore_barrier()
        run_pass(ei01_hbm, 0, 1, x0_hbm)
        plsc.subcore_barrier()
        dump(2)

    return k(x0, x1, ei00, ei01, zstrip)


def _combine(x0, x1, P, W0, W1, b0, b1):
    BR = 1000
    grid = (N // BR,)

    def body(x0_ref, x1_ref, p_ref, w0_ref, w1_ref, b0_ref, b1_ref,
             o0_ref, o1_ref):
        p = p_ref[...]
        cum0 = p[0, 0] + p[0, 1]
        cum1 = p[1, 0] + p[1, 1]
        cum2 = p[2, 0] + p[2, 1]
        a00 = cum0
        a01 = cum1 - cum0
        a10 = cum2 - cum1
        w0 = w0_ref[...]
        w1 = w1_ref[...]
        u0 = x0_ref[...] + a00
        o0_ref[...] = (
            jnp.dot(u0, w0, preferred_element_type=jnp.float32)
            + jnp.dot(a01, w1, preferred_element_type=jnp.float32)
            + 3.0 * b0_ref[...] + b1_ref[...]
        )
        o1_ref[...] = (
            jnp.dot(x1_ref[...], w1, preferred_element_type=jnp.float32)
            + jnp.dot(a10, w0, preferred_element_type=jnp.float32)
            + b0_ref[...] + b1_ref[...]
        )

    return pl.pallas_call(
        body,
        grid=grid,
        in_specs=[
            pl.BlockSpec((BR, D), lambda i: (i, 0)),
            pl.BlockSpec((BR, D), lambda i: (i, 0)),
            pl.BlockSpec((3, NC, BR, D), lambda i: (0, 0, i, 0)),
            pl.BlockSpec((D, D), lambda i: (0, 0)),
            pl.BlockSpec((D, D), lambda i: (0, 0)),
            pl.BlockSpec((1, D), lambda i: (0, 0)),
            pl.BlockSpec((1, D), lambda i: (0, 0)),
        ],
        out_specs=[
            pl.BlockSpec((BR, D), lambda i: (i, 0)),
            pl.BlockSpec((BR, D), lambda i: (i, 0)),
        ],
        out_shape=[
            jax.ShapeDtypeStruct((N, D), jnp.float32),
            jax.ShapeDtypeStruct((N, D), jnp.float32),
        ],
    )(x0, x1, P, W0, W1, b0.reshape(1, D), b1.reshape(1, D))


def kernel(x0, x1, edge_index_00, edge_index_01, W0, b0, W1, b1):
    zstrip = jnp.zeros((STRIP, D), jnp.float32)
    P = _sc_scatter(x0, x1, edge_index_00.reshape(-1),
                    edge_index_01.reshape(-1), zstrip)
    out0, out1 = _combine(x0, x1, P, W0, W1, b0, b1)
    return out0, out1
